# baseline (speedup 1.0000x reference)
import functools

import jax
import jax.numpy as jnp
from jax.experimental import pallas as pl
from jax.experimental.pallas import tpu as pltpu

_EPS = 1e-8
_INF = float("inf")


def _chamfer_body(p_ref, g_ref, out_ref, bsel_ref, bval_ref, facc_ref,
                  bacc_ref, *, inv_bn, inv_bm):
    b = pl.program_id(0)
    n = pl.program_id(1)
    nb = pl.num_programs(1)
    blast = pl.num_programs(0) - 1

    p = p_ref[0]
    g = g_ref[0]
    p2 = jnp.sum(p * p, axis=1, keepdims=True)
    g2 = jnp.sum(g * g, axis=0, keepdims=True)

    g_hi = g.astype(jnp.bfloat16)
    pm2_hi = (-2.0 * p).astype(jnp.bfloat16)
    crossm2 = jax.lax.dot_general(
        pm2_hi, g_hi, (((1,), (0,)), ((), ())),
        preferred_element_type=jnp.float32)
    d_sel = (p2 + crossm2) + g2

    pm2 = -2.0 * p
    pm2_lo = (pm2 - pm2_hi.astype(jnp.float32)).astype(jnp.bfloat16)
    g_lo = (g - g_hi.astype(jnp.float32)).astype(jnp.bfloat16)
    corrm2 = jax.lax.dot_general(
        jnp.concatenate([pm2_lo, pm2_hi], axis=1),
        jnp.concatenate([g_hi, g_lo], axis=0),
        (((1,), (0,)), ((), ())),
        preferred_element_type=jnp.float32)

    @pl.when(jnp.logical_and(b == 0, n == 0))
    def _init_acc():
        facc_ref[...] = jnp.zeros_like(facc_ref)
        bacc_ref[...] = jnp.zeros_like(bacc_ref)

    fsel = jnp.min(d_sel, axis=1, keepdims=True)
    fcorr = jnp.min(jnp.where(d_sel == fsel, corrm2, _INF),
                    axis=1, keepdims=True)
    fval = fsel + fcorr
    fsum = jnp.sum(jnp.sqrt(jnp.maximum(fval, 0.0) + _EPS), keepdims=True)
    facc_ref[...] = facc_ref[...] + fsum.reshape(1, 1)

    csel = jnp.min(d_sel, axis=0, keepdims=True)
    cval = jnp.min(jnp.where(d_sel == csel, corrm2, _INF),
                   axis=0, keepdims=True)

    @pl.when(n == 0)
    def _bwd_init():
        bsel_ref[...] = csel
        bval_ref[...] = cval

    @pl.when(n > 0)
    def _bwd_acc():
        rsel = bsel_ref[...]
        rval = bval_ref[...]
        bval_ref[...] = jnp.where(
            csel < rsel, cval,
            jnp.where(csel == rsel, jnp.minimum(cval, rval), rval))
        bsel_ref[...] = jnp.minimum(rsel, csel)

    @pl.when(n == nb - 1)
    def _bwd_done():
        bval = bsel_ref[...] + bval_ref[...]
        bsum = jnp.sum(jnp.sqrt(jnp.maximum(bval, 0.0) + _EPS),
                       keepdims=True)
        bacc_ref[...] = bacc_ref[...] + bsum.reshape(1, 1)

    @pl.when(jnp.logical_and(b == blast, n == nb - 1))
    def _finalize():
        out_ref[...] = facc_ref[...] * inv_bn + bacc_ref[...] * inv_bm


def kernel(predict_pc, gt_pc):
    B, C, N = predict_pc.shape
    M = gt_pc.shape[2]
    NT = 256
    pT = jnp.transpose(predict_pc, (0, 2, 1))

    body = functools.partial(_chamfer_body,
                             inv_bn=1.0 / (B * N), inv_bm=1.0 / (B * M))
    out = pl.pallas_call(
        body,
        grid=(B, N // NT),
        in_specs=[
            pl.BlockSpec((1, NT, C), lambda b, n: (b, n, 0)),
            pl.BlockSpec((1, C, M), lambda b, n: (b, 0, 0)),
        ],
        out_specs=pl.BlockSpec((1, 1), lambda b, n: (0, 0)),
        out_shape=jax.ShapeDtypeStruct((1, 1), jnp.float32),
        scratch_shapes=[
            pltpu.VMEM((1, M), jnp.float32),
            pltpu.VMEM((1, M), jnp.float32),
            pltpu.VMEM((1, 1), jnp.float32),
            pltpu.VMEM((1, 1), jnp.float32),
        ],
    )(pT, gt_pc)
    return out[0, 0]


# speedup vs baseline: 1.0403x; 1.0403x over previous
import functools

import jax
import jax.numpy as jnp
from jax.experimental import pallas as pl
from jax.experimental.pallas import tpu as pltpu

_EPS = 1e-8
_INF = float("inf")


def _chamfer_body(p_ref, g_ref, out_ref, bsel_ref, bval_ref, facc_ref,
                  bacc_ref, *, inv_bn, inv_bm):
    b = pl.program_id(0)
    n = pl.program_id(1)
    nb = pl.num_programs(1)
    blast = pl.num_programs(0) - 1

    p = p_ref[0]
    g = g_ref[0]
    p2 = jnp.sum(p * p, axis=1, keepdims=True)
    g2 = jnp.sum(g * g, axis=0, keepdims=True)

    g_hi = g.astype(jnp.bfloat16)
    pm2_hi = (-2.0 * p).astype(jnp.bfloat16)
    crossm2 = jax.lax.dot_general(
        pm2_hi, g_hi, (((1,), (0,)), ((), ())),
        preferred_element_type=jnp.float32)
    d_sel = (p2 + crossm2) + g2

    pm2 = -2.0 * p
    pm2_lo = (pm2 - pm2_hi.astype(jnp.float32)).astype(jnp.bfloat16)
    g_lo = (g - g_hi.astype(jnp.float32)).astype(jnp.bfloat16)
    corrm2 = jax.lax.dot_general(
        jnp.concatenate([pm2_lo, pm2_hi], axis=1),
        jnp.concatenate([g_hi, g_lo], axis=0),
        (((1,), (0,)), ((), ())),
        preferred_element_type=jnp.float32)

    @pl.when(jnp.logical_and(b == 0, n == 0))
    def _init_acc():
        facc_ref[...] = jnp.zeros_like(facc_ref)
        bacc_ref[...] = jnp.zeros_like(bacc_ref)

    fsel = jnp.min(d_sel, axis=1, keepdims=True)
    fcorr = jnp.min(jnp.where(d_sel == fsel, corrm2, _INF),
                    axis=1, keepdims=True)
    fval = fsel + fcorr
    fsum = jnp.sum(jnp.sqrt(jnp.maximum(fval, 0.0) + _EPS), keepdims=True)
    facc_ref[...] = facc_ref[...] + fsum.reshape(1, 1)

    csel = jnp.min(d_sel, axis=0, keepdims=True)
    cval = jnp.min(jnp.where(d_sel == csel, corrm2, _INF),
                   axis=0, keepdims=True)

    @pl.when(n == 0)
    def _bwd_init():
        bsel_ref[...] = csel
        bval_ref[...] = cval

    @pl.when(n > 0)
    def _bwd_acc():
        rsel = bsel_ref[...]
        rval = bval_ref[...]
        bval_ref[...] = jnp.where(
            csel < rsel, cval,
            jnp.where(csel == rsel, jnp.minimum(cval, rval), rval))
        bsel_ref[...] = jnp.minimum(rsel, csel)

    @pl.when(n == nb - 1)
    def _bwd_done():
        bval = bsel_ref[...] + bval_ref[...]
        bsum = jnp.sum(jnp.sqrt(jnp.maximum(bval, 0.0) + _EPS),
                       keepdims=True)
        bacc_ref[...] = bacc_ref[...] + bsum.reshape(1, 1)

    @pl.when(jnp.logical_and(b == blast, n == nb - 1))
    def _finalize():
        out_ref[...] = facc_ref[...] * inv_bn + bacc_ref[...] * inv_bm


def kernel(predict_pc, gt_pc):
    B, C, N = predict_pc.shape
    M = gt_pc.shape[2]
    NT = 512
    pT = jnp.transpose(predict_pc, (0, 2, 1))

    body = functools.partial(_chamfer_body,
                             inv_bn=1.0 / (B * N), inv_bm=1.0 / (B * M))
    out = pl.pallas_call(
        body,
        grid=(B, N // NT),
        in_specs=[
            pl.BlockSpec((1, NT, C), lambda b, n: (b, n, 0)),
            pl.BlockSpec((1, C, M), lambda b, n: (b, 0, 0)),
        ],
        out_specs=pl.BlockSpec((1, 1), lambda b, n: (0, 0)),
        out_shape=jax.ShapeDtypeStruct((1, 1), jnp.float32),
        scratch_shapes=[
            pltpu.VMEM((1, M), jnp.float32),
            pltpu.VMEM((1, M), jnp.float32),
            pltpu.VMEM((1, 1), jnp.float32),
            pltpu.VMEM((1, 1), jnp.float32),
        ],
    )(pT, gt_pc)
    return out[0, 0]
